# baseline (speedup 1.0000x reference)
import functools

import jax
import jax.numpy as jnp
from jax import lax
from jax.experimental import pallas as pl
from jax.experimental.pallas import tpu as pltpu
from jax.experimental.pallas import tpu_sc as plsc

_N = 10000
_E = 320000
_D = 128
_GAMMA = 0.5

_NC = 2
_NS = 16
_NW = _NC * _NS
_L = 16
_EPW = _E // _NW
_U = 5
_EW = _EPW + 112
_MVB = 2048
_NP = 10240


def _mv_body(b_ref, d_ref, x_ref, o_ref, s_ref):
    wt = jnp.stack([b_ref[...], d_ref[...]], axis=0)
    m = lax.dot_general(
        wt, x_ref[...], (((1,), (1,)), ((), ())),
        preferred_element_type=jnp.float32,
        precision=lax.Precision.HIGHEST,
    )
    o_ref[...] = m
    s_ref[...] = m[1, :]


def _matvecs(beta, delta, x):
    return pl.pallas_call(
        _mv_body,
        grid=(_NP // _MVB,),
        in_specs=[
            pl.BlockSpec((_D,), lambda i: (0,)),
            pl.BlockSpec((_D,), lambda i: (0,)),
            pl.BlockSpec((_MVB, _D), lambda i: (i, 0)),
        ],
        out_specs=[
            pl.BlockSpec((2, _MVB), lambda i: (0, i)),
            pl.BlockSpec((_MVB,), lambda i: (i,)),
        ],
        out_shape=[
            jax.ShapeDtypeStruct((2, _NP), jnp.float32),
            jax.ShapeDtypeStruct((_NP,), jnp.float32),
        ],
    )(beta, delta, x)


_mesh = plsc.VectorSubcoreMesh(core_axis_name="c", subcore_axis_name="s")


def _worker_window(ei_hbm, ed_v, sem):
    cid = lax.axis_index("c")
    sid = lax.axis_index("s")
    wid = cid * _NS + sid
    base = wid * _EPW
    abase = (base // 128) * 128
    off0 = base - abase
    cp = pltpu.async_copy(ei_hbm.at[:, pl.ds(abase, _EW)], ed_v, sem)
    return wid, off0, cp


@functools.partial(
    pl.kernel,
    mesh=_mesh,
    compiler_params=pltpu.CompilerParams(needs_layout_passes=False),
    out_type=[
        jax.ShapeDtypeStruct((_NW, _N), jnp.float32),
        jax.ShapeDtypeStruct((_NW, _L), jnp.int32),
    ],
    scratch_types=[
        pltpu.VMEM((2, _EW), jnp.int32),
        pltpu.VMEM((_N,), jnp.float32),
        pltpu.VMEM((_L,), jnp.int32),
        pltpu.SemaphoreType.DMA,
    ],
)
def _count_kernel(ei_hbm, c_out, m_out, ed_v, c_v, m_v, sem0):
    wid, off0, cp0 = _worker_window(ei_hbm, ed_v, sem0)

    zeros = jnp.zeros((_L,), jnp.float32)

    def zero_body(i, carry):
        for j in range(_U):
            z = pl.multiple_of((i * _U + j) * _L, _L)
            c_v[pl.ds(z, _L)] = zeros
        return carry

    lax.fori_loop(0, _N // _L // _U, zero_body, 0)
    cp0.wait()

    ones = jnp.ones((_L,), jnp.float32)

    def edge_body(i, m):
        svs = []
        dvs = []
        for j in range(_U):
            off = pl.multiple_of(off0 + (i * _U + j) * _L, _L)
            svs.append(ed_v[0, pl.ds(off, _L)])
            dvs.append(ed_v[1, pl.ds(off, _L)])
        for j in range(_U):
            plsc.addupdate_scatter(c_v, [dvs[j]], ones)
        for j in range(_U):
            m = jnp.maximum(m, svs[j])
        return m

    m16 = lax.fori_loop(0, _EPW // _L // _U, edge_body,
                        jnp.zeros((_L,), jnp.int32))
    m_v[...] = m16

    wp0 = pltpu.async_copy(c_v, c_out.at[wid], sem0)
    wp0.wait()
    pltpu.sync_copy(m_v, m_out.at[wid])


@functools.partial(
    pl.kernel,
    mesh=_mesh,
    compiler_params=pltpu.CompilerParams(needs_layout_passes=False),
    out_type=jax.ShapeDtypeStruct((_NW, _N), jnp.float32),
    scratch_types=[
        pltpu.VMEM((2, _EW), jnp.int32),
        pltpu.VMEM((_NP,), jnp.float32),
        pltpu.VMEM((_N,), jnp.float32),
        pltpu.SemaphoreType.DMA,
        pltpu.SemaphoreType.DMA,
    ],
)
def _gather_kernel(ei_hbm, s_hbm, t_out, ed_v, s_v, t_v, sem0, sem1):
    wid, off0, cp0 = _worker_window(ei_hbm, ed_v, sem0)
    cp1 = pltpu.async_copy(s_hbm, s_v, sem1)

    zeros = jnp.zeros((_L,), jnp.float32)

    def zero_body(i, carry):
        for j in range(_U):
            z = pl.multiple_of((i * _U + j) * _L, _L)
            t_v[pl.ds(z, _L)] = zeros
        return carry

    lax.fori_loop(0, _N // _L // _U, zero_body, 0)
    cp0.wait()
    cp1.wait()

    def edge_body(i, carry):
        svs = []
        dvs = []
        for j in range(_U):
            off = pl.multiple_of(off0 + (i * _U + j) * _L, _L)
            svs.append(ed_v[0, pl.ds(off, _L)])
            dvs.append(ed_v[1, pl.ds(off, _L)])
        gs = [plsc.load_gather(s_v, [sv]) for sv in svs]
        for j in range(_U):
            plsc.addupdate_scatter(t_v, [dvs[j]], gs[j])
        return carry

    lax.fori_loop(0, _EPW // _L // _U, edge_body, 0)

    wp0 = pltpu.async_copy(t_v, t_out.at[wid], sem0)
    wp0.wait()


def _final_body(mt_ref, t_ref, c_ref, mx_ref, o_ref):
    t = jnp.sum(t_ref[...], axis=0, keepdims=True)
    c = jnp.sum(c_ref[...], axis=0, keepdims=True)
    mx = jnp.max(mx_ref[...]).astype(jnp.float32)
    bc = mt_ref[0:1, :_N]
    logits = bc + (_GAMMA * (mx + 1.0) / _E) * c + t / jnp.maximum(c, 1.0)
    mean = jnp.sum(logits) / _N
    d = logits - mean
    var = jnp.sum(d * d) / (_N - 1)
    y = jax.nn.sigmoid(lax.rsqrt(var) * d)
    o_ref[...] = jnp.concatenate([1.0 - y, y], axis=0)


def _finale(mt, tpart, cpart, mxpart):
    return pl.pallas_call(
        _final_body,
        out_shape=jax.ShapeDtypeStruct((2, _N), jnp.float32),
    )(mt, tpart, cpart, mxpart)


def kernel(x, edge_index, beta, delta):
    cpart, mxpart = _count_kernel(edge_index)
    mt, s = _matvecs(beta, delta, x)
    tpart = _gather_kernel(edge_index, s)
    return _finale(mt, tpart, cpart, mxpart).T


# speedup vs baseline: 1.1414x; 1.1414x over previous
import functools

import jax
import jax.numpy as jnp
from jax import lax
from jax.experimental import pallas as pl
from jax.experimental.pallas import tpu as pltpu
from jax.experimental.pallas import tpu_sc as plsc

_N = 10000
_E = 320000
_D = 128
_GAMMA = 0.5

_NC = 2
_NS = 16
_NW = _NC * _NS
_L = 16
_EPW = _E // _NW
_U = 25
_EW = _EPW + 112
_MVB = 2048
_NP = 10240


def _mv_body(b_ref, d_ref, x_ref, o_ref, s_ref):
    wt = jnp.stack([b_ref[...], d_ref[...]], axis=0)
    m = lax.dot_general(
        wt, x_ref[...], (((1,), (1,)), ((), ())),
        preferred_element_type=jnp.float32,
        precision=lax.Precision.HIGHEST,
    )
    o_ref[...] = m
    s_ref[...] = m[1, :]


def _matvecs(beta, delta, x):
    return pl.pallas_call(
        _mv_body,
        grid=(_NP // _MVB,),
        in_specs=[
            pl.BlockSpec((_D,), lambda i: (0,)),
            pl.BlockSpec((_D,), lambda i: (0,)),
            pl.BlockSpec((_MVB, _D), lambda i: (i, 0)),
        ],
        out_specs=[
            pl.BlockSpec((2, _MVB), lambda i: (0, i)),
            pl.BlockSpec((_MVB,), lambda i: (i,)),
        ],
        out_shape=[
            jax.ShapeDtypeStruct((2, _NP), jnp.float32),
            jax.ShapeDtypeStruct((_NP,), jnp.float32),
        ],
    )(beta, delta, x)


_mesh = plsc.VectorSubcoreMesh(core_axis_name="c", subcore_axis_name="s")


@functools.partial(
    pl.kernel,
    mesh=_mesh,
    compiler_params=pltpu.CompilerParams(needs_layout_passes=False),
    out_type=[
        jax.ShapeDtypeStruct((_NW, _N), jnp.float32),
        jax.ShapeDtypeStruct((_NW, _N), jnp.float32),
        jax.ShapeDtypeStruct((_NW, _L), jnp.int32),
    ],
    scratch_types=[
        pltpu.VMEM((2, _EW), jnp.int32),
        pltpu.VMEM((_NP,), jnp.float32),
        pltpu.VMEM((_N,), jnp.float32),
        pltpu.VMEM((_N,), jnp.float32),
        pltpu.VMEM((_L,), jnp.int32),
        pltpu.SemaphoreType.DMA,
        pltpu.SemaphoreType.DMA,
    ],
)
def _edge_kernel(ei_hbm, s_hbm, t_out, c_out, m_out,
                 ed_v, s_v, t_v, c_v, m_v, sem0, sem1):
    cid = lax.axis_index("c")
    sid = lax.axis_index("s")
    wid = cid * _NS + sid
    base = wid * _EPW
    abase = (base // 128) * 128
    off0 = base - abase

    cp0 = pltpu.async_copy(ei_hbm.at[:, pl.ds(abase, _EW)], ed_v, sem0)
    cp1 = pltpu.async_copy(s_hbm, s_v, sem1)

    zeros = jnp.zeros((_L,), jnp.float32)

    def zero_body(i, carry):
        for j in range(_U):
            z = pl.multiple_of((i * _U + j) * _L, _L)
            t_v[pl.ds(z, _L)] = zeros
            c_v[pl.ds(z, _L)] = zeros
        return carry

    lax.fori_loop(0, _N // _L // _U, zero_body, 0)

    cp0.wait()
    cp1.wait()

    ones = jnp.ones((_L,), jnp.float32)

    def edge_body(i, m):
        svs = []
        dvs = []
        for j in range(_U):
            off = pl.multiple_of(off0 + (i * _U + j) * _L, _L)
            svs.append(ed_v[0, pl.ds(off, _L)])
            dvs.append(ed_v[1, pl.ds(off, _L)])
        gs = [plsc.load_gather(s_v, [sv]) for sv in svs]
        for j in range(_U):
            plsc.addupdate_scatter(t_v, [dvs[j]], gs[j])
            plsc.addupdate_scatter(c_v, [dvs[j]], ones)
        for j in range(_U):
            m = jnp.maximum(m, svs[j])
        return m

    m16 = lax.fori_loop(0, _EPW // _L // _U, edge_body,
                        jnp.zeros((_L,), jnp.int32))
    m_v[...] = m16

    wp0 = pltpu.async_copy(t_v, t_out.at[wid], sem0)
    wp1 = pltpu.async_copy(c_v, c_out.at[wid], sem1)
    wp0.wait()
    wp1.wait()
    pltpu.sync_copy(m_v, m_out.at[wid])


def _final_body(mt_ref, t_ref, c_ref, mx_ref, o_ref):
    t = jnp.sum(t_ref[...], axis=0, keepdims=True)
    c = jnp.sum(c_ref[...], axis=0, keepdims=True)
    mx = jnp.max(mx_ref[...]).astype(jnp.float32)
    bc = mt_ref[0:1, :_N]
    logits = bc + (_GAMMA * (mx + 1.0) / _E) * c + t / jnp.maximum(c, 1.0)
    mean = jnp.sum(logits) / _N
    d = logits - mean
    var = jnp.sum(d * d) / (_N - 1)
    y = jax.nn.sigmoid(lax.rsqrt(var) * d)
    o_ref[...] = jnp.concatenate([1.0 - y, y], axis=0)


def _finale(mt, tpart, cpart, mxpart):
    return pl.pallas_call(
        _final_body,
        out_shape=jax.ShapeDtypeStruct((2, _N), jnp.float32),
    )(mt, tpart, cpart, mxpart)


def kernel(x, edge_index, beta, delta):
    mt, s = _matvecs(beta, delta, x)
    tpart, cpart, mxpart = _edge_kernel(edge_index, s)
    return _finale(mt, tpart, cpart, mxpart).T


# speedup vs baseline: 1.1719x; 1.0267x over previous
import functools

import jax
import jax.numpy as jnp
from jax import lax
from jax.experimental import pallas as pl
from jax.experimental.pallas import tpu as pltpu
from jax.experimental.pallas import tpu_sc as plsc

_N = 10000
_E = 320000
_D = 128
_GAMMA = 0.5

_NC = 2
_NS = 16
_NW = _NC * _NS
_L = 16
_EPW = _E // _NW
_U = 5
_EW = _EPW + 112
_MVB = 2048
_NP = 10240


def _mv_body(b_ref, d_ref, x_ref, o_ref, s_ref):
    wt = jnp.stack([b_ref[...], d_ref[...]], axis=0)
    m = lax.dot_general(
        wt, x_ref[...], (((1,), (1,)), ((), ())),
        preferred_element_type=jnp.float32,
        precision=lax.Precision.HIGHEST,
    )
    o_ref[...] = m
    s_ref[...] = m[1, :]


def _matvecs(beta, delta, x):
    return pl.pallas_call(
        _mv_body,
        grid=(_NP // _MVB,),
        in_specs=[
            pl.BlockSpec((_D,), lambda i: (0,)),
            pl.BlockSpec((_D,), lambda i: (0,)),
            pl.BlockSpec((_MVB, _D), lambda i: (i, 0)),
        ],
        out_specs=[
            pl.BlockSpec((2, _MVB), lambda i: (0, i)),
            pl.BlockSpec((_MVB,), lambda i: (i,)),
        ],
        out_shape=[
            jax.ShapeDtypeStruct((2, _NP), jnp.float32),
            jax.ShapeDtypeStruct((_NP,), jnp.float32),
        ],
    )(beta, delta, x)


_mesh = plsc.VectorSubcoreMesh(core_axis_name="c", subcore_axis_name="s")


@functools.partial(
    pl.kernel,
    mesh=_mesh,
    compiler_params=pltpu.CompilerParams(needs_layout_passes=False),
    out_type=[
        jax.ShapeDtypeStruct((_NW, _N), jnp.float32),
        jax.ShapeDtypeStruct((_NW, _N), jnp.float32),
        jax.ShapeDtypeStruct((_NW, _L), jnp.int32),
    ],
    scratch_types=[
        pltpu.VMEM((2, _EW), jnp.int32),
        pltpu.VMEM((_NP,), jnp.float32),
        pltpu.VMEM((_N,), jnp.float32),
        pltpu.VMEM((_N,), jnp.float32),
        pltpu.VMEM((_L,), jnp.int32),
        pltpu.SemaphoreType.DMA,
        pltpu.SemaphoreType.DMA,
    ],
)
def _edge_kernel(ei_hbm, s_hbm, t_out, c_out, m_out,
                 ed_v, s_v, t_v, c_v, m_v, sem0, sem1):
    cid = lax.axis_index("c")
    sid = lax.axis_index("s")
    wid = cid * _NS + sid
    base = wid * _EPW
    abase = (base // 128) * 128
    off0 = base - abase

    cp0 = pltpu.async_copy(ei_hbm.at[:, pl.ds(abase, _EW)], ed_v, sem0)
    cp1 = pltpu.async_copy(s_hbm, s_v, sem1)

    zeros = jnp.zeros((_L,), jnp.float32)

    def zero_body(i, carry):
        for j in range(_U):
            z = pl.multiple_of((i * _U + j) * _L, _L)
            t_v[pl.ds(z, _L)] = zeros
            c_v[pl.ds(z, _L)] = zeros
        return carry

    lax.fori_loop(0, _N // _L // _U, zero_body, 0)

    cp0.wait()
    cp1.wait()

    ones = jnp.ones((_L,), jnp.float32)

    def edge_body(i, m):
        svs = []
        dvs = []
        for j in range(_U):
            off = pl.multiple_of(off0 + (i * _U + j) * _L, _L)
            svs.append(ed_v[0, pl.ds(off, _L)])
            dvs.append(ed_v[1, pl.ds(off, _L)])
        gs = [plsc.load_gather(s_v, [sv]) for sv in svs]
        for j in range(_U):
            plsc.addupdate_scatter(t_v, [dvs[j]], gs[j])
        for j in range(_U):
            plsc.addupdate_scatter(c_v, [dvs[j]], ones)
        for j in range(_U):
            m = jnp.maximum(m, svs[j])
        return m

    m16 = lax.fori_loop(0, _EPW // _L // _U, edge_body,
                        jnp.zeros((_L,), jnp.int32))
    m_v[...] = m16

    wp0 = pltpu.async_copy(t_v, t_out.at[wid], sem0)
    wp1 = pltpu.async_copy(c_v, c_out.at[wid], sem1)
    wp0.wait()
    wp1.wait()
    pltpu.sync_copy(m_v, m_out.at[wid])


def _final_body(mt_ref, t_ref, c_ref, mx_ref, o_ref):
    t = jnp.sum(t_ref[...], axis=0, keepdims=True)
    c = jnp.sum(c_ref[...], axis=0, keepdims=True)
    mx = jnp.max(mx_ref[...]).astype(jnp.float32)
    bc = mt_ref[0:1, :_N]
    logits = bc + (_GAMMA * (mx + 1.0) / _E) * c + t / jnp.maximum(c, 1.0)
    mean = jnp.sum(logits) / _N
    d = logits - mean
    var = jnp.sum(d * d) / (_N - 1)
    y = jax.nn.sigmoid(lax.rsqrt(var) * d)
    o_ref[...] = jnp.concatenate([1.0 - y, y], axis=0)


def _finale(mt, tpart, cpart, mxpart):
    return pl.pallas_call(
        _final_body,
        out_shape=jax.ShapeDtypeStruct((2, _N), jnp.float32),
    )(mt, tpart, cpart, mxpart)


def kernel(x, edge_index, beta, delta):
    mt, s = _matvecs(beta, delta, x)
    tpart, cpart, mxpart = _edge_kernel(edge_index, s)
    return _finale(mt, tpart, cpart, mxpart).T
